# Initial kernel scaffold; baseline (speedup 1.0000x reference)
import jax
import jax.numpy as jnp
from jax.experimental import pallas as pl


def kernel(xn, xn_attr, xe_attr, xe_src, xe_dst, Ms_M, Ms_w, fc1_w, fc1_b, fc2_w, fc2_b, n2e_M1, n2e_M2, mvv_M1, mvv_M2, e2n_M1, e2n_M2):
    raise NotImplementedError("write your pallas kernel here")


# speedup vs baseline: 1.0545x; 1.0545x over previous
import functools

import jax
import jax.numpy as jnp
from jax.experimental import pallas as pl
from jax.experimental.pallas import tpu as pltpu

N = 100000
E = 1600000
D = 3
V = 16

_BN = 2000
_BE2 = 6400


def _silu(x):
    return x * jax.nn.sigmoid(x)


def _premix_body(cw_ref, xn_ref, attr_ref, msm_ref, msmT_ref, out_ref):
    xn = xn_ref[...]
    attr = attr_ref[...]
    bn = xn.shape[0]
    a = xn.reshape(bn * D, V)
    y = jnp.dot(a, msm_ref[...], preferred_element_type=jnp.float32)
    y = y.reshape(bn, D, V) * attr[:, None, :]
    z = jnp.dot(y.reshape(bn * D, V), msmT_ref[...],
                preferred_element_type=jnp.float32).reshape(bn, D, V)
    out_ref[...] = cw_ref[0] * xn + cw_ref[1] * z


def _premix(xn, xn_attr, Ms_M, cw):
    return pl.pallas_call(
        _premix_body,
        grid=(N // _BN,),
        in_specs=[
            pl.BlockSpec(memory_space=pltpu.SMEM),
            pl.BlockSpec((_BN, D, V), lambda i: (i, 0, 0)),
            pl.BlockSpec((_BN, V), lambda i: (i, 0)),
            pl.BlockSpec((V, V), lambda i: (0, 0)),
            pl.BlockSpec((V, V), lambda i: (0, 0)),
        ],
        out_specs=pl.BlockSpec((_BN, D, V), lambda i: (i, 0, 0)),
        out_shape=jax.ShapeDtypeStruct((N, D, V), jnp.float32),
    )(cw, xn, xn_attr, Ms_M, Ms_M.T)


def _edge_body(attr_ref, gsd_ref, f1_ref, b1_ref, f2_ref, b2_ref, w_ref,
               out_ref):
    a0 = attr_ref[:, 0:1]
    a1 = attr_ref[:, 1:2]
    w1 = jnp.concatenate(
        [_silu(a0 * f1_ref[...] + b1_ref[...]),
         _silu(a1 * f1_ref[...] + b1_ref[...])], axis=1)
    ws = gsd_ref[...] * w1
    mm = jnp.dot(ws, w_ref[...], preferred_element_type=jnp.float32)
    w2 = jnp.concatenate(
        [_silu(a0 * f2_ref[...] + b2_ref[...]),
         _silu(a1 * f2_ref[...] + b2_ref[...])], axis=1)
    out_ref[...] = mm * w2


def _edge(attr2, gsd, f1t, b1t, f2t, b2t, w192):
    return pl.pallas_call(
        _edge_body,
        grid=(E // 2 // _BE2,),
        in_specs=[
            pl.BlockSpec((_BE2, 2), lambda i: (i, 0)),
            pl.BlockSpec((_BE2, 192), lambda i: (i, 0)),
            pl.BlockSpec((1, 96), lambda i: (0, 0)),
            pl.BlockSpec((1, 96), lambda i: (0, 0)),
            pl.BlockSpec((1, 48), lambda i: (0, 0)),
            pl.BlockSpec((1, 48), lambda i: (0, 0)),
            pl.BlockSpec((192, 96), lambda i: (0, 0)),
        ],
        out_specs=pl.BlockSpec((_BE2, 96), lambda i: (i, 0)),
        out_shape=jax.ShapeDtypeStruct((E // 2, 96), jnp.float32),
    )(attr2, gsd, f1t, b1t, f2t, b2t, w192)


def _final_body(a1_ref, a2_ref, cd_ref, out_ref):
    a1 = a1_ref[...]
    a2 = a2_ref[...]
    bn = a1.shape[0]
    t = (jnp.dot(a1.reshape(bn * D, V), cd_ref[0:V, :],
                 preferred_element_type=jnp.float32)
         + jnp.dot(a2.reshape(bn * D, V), cd_ref[V:2 * V, :],
                   preferred_element_type=jnp.float32)).reshape(bn, D, V)
    nrm = jnp.sqrt(jnp.sum(t * t, axis=1))
    out_ref[...] = t * jnp.tanh(nrm)[:, None, :]


def _final(acc1, acc2, cd):
    return pl.pallas_call(
        _final_body,
        grid=(N // _BN,),
        in_specs=[
            pl.BlockSpec((_BN, D, V), lambda i: (i, 0, 0)),
            pl.BlockSpec((_BN, D, V), lambda i: (i, 0, 0)),
            pl.BlockSpec((2 * V, V), lambda i: (0, 0)),
        ],
        out_specs=pl.BlockSpec((_BN, D, V), lambda i: (i, 0, 0)),
        out_shape=jax.ShapeDtypeStruct((N, D, V), jnp.float32),
    )(acc1, acc2, cd)


def kernel(xn, xn_attr, xe_attr, xe_src, xe_dst, Ms_M, Ms_w, fc1_w, fc1_b,
           fc2_w, fc2_b, n2e_M1, n2e_M2, mvv_M1, mvv_M2, e2n_M1, e2n_M2):
    ang = 0.1 * Ms_w[0]
    cw = jnp.stack([jnp.cos(ang), jnp.sin(ang)])
    Rm = (mvv_M1 + mvv_M2) * 0.5
    A = (n2e_M1 * 0.5 + n2e_M2 * 0.25) @ Rm
    B = (-n2e_M1 * 0.5 + n2e_M2 * 0.25) @ Rm
    eye3 = jnp.eye(3, dtype=jnp.float32)
    bda = jnp.kron(eye3, A)
    bdb = jnp.kron(eye3, B)
    w96 = jnp.concatenate([bda, bdb], axis=0)
    w192 = jnp.kron(jnp.eye(2, dtype=jnp.float32), w96)
    cd = jnp.concatenate([(e2n_M1 + e2n_M2) * 0.5,
                          (e2n_M2 - e2n_M1) * 0.5], axis=0)
    f1t = jnp.tile(fc1_w[:, 0], 6)[None, :]
    b1t = jnp.tile(fc1_b, 6)[None, :]
    f2t = jnp.tile(fc2_w[:, 0], 3)[None, :]
    b2t = jnp.tile(fc2_b, 3)[None, :]

    xnp = _premix(xn, xn_attr, Ms_M, cw).reshape(N, D * V)

    ii = jnp.stack([xe_src, xe_dst], axis=1).reshape(2 * E)
    gsd = xnp[ii].reshape(E // 2, 4 * D * V)
    m2 = _edge(xe_attr.reshape(E // 2, 2), gsd, f1t, b1t, f2t, b2t, w192)

    m = m2.reshape(E, D, V)
    acc1 = jax.ops.segment_sum(m, xe_dst, num_segments=N)
    acc2 = jax.ops.segment_sum(m, xe_src, num_segments=N)

    return _final(acc1, acc2, cd)


# speedup vs baseline: 1.0861x; 1.0300x over previous
import functools

import jax
import jax.numpy as jnp
from jax import lax
from jax.experimental import pallas as pl
from jax.experimental.pallas import tpu as pltpu
from jax.experimental.pallas import tpu_sc as plsc

N = 100000
E = 1600000
D = 3
V = 16

_BN = 3128
_BE2 = 6400

_EP = 1638400
_IW = 2 * _EP // 128
_WPW = _IW // 32
_GU = 8

_mesh = plsc.VectorSubcoreMesh(core_axis_name="c", subcore_axis_name="s")

_NP = 100096


@functools.partial(
    pl.kernel,
    out_type=jax.ShapeDtypeStruct((_NP, 48), jnp.float32),
    mesh=_mesh,
    scratch_types=[pltpu.VMEM((184, 48), jnp.float32)],
    compiler_params=pltpu.CompilerParams(use_tc_tiling_on_sc=False),
)
def _relayout_sc(src_hbm, out_hbm, buf):
    wid = lax.axis_index("s") * 2 + lax.axis_index("c")
    base = wid * 3128

    def step(k, carry):
        r0 = base + k * 184
        pltpu.sync_copy(src_hbm.at[pl.ds(r0, 184)], buf)
        pltpu.sync_copy(buf, out_hbm.at[pl.ds(r0, 184)])
        return carry

    lax.fori_loop(0, 17, step, 0)


@functools.partial(
    pl.kernel,
    out_type=jax.ShapeDtypeStruct((_IW, 128, 48), jnp.float32),
    mesh=_mesh,
    scratch_types=[
        pltpu.VMEM((_GU, 128), jnp.int32),
        pltpu.VMEM((_GU, 128, 48), jnp.float32),
        pltpu.SemaphoreType.DMA,
    ],
    compiler_params=pltpu.CompilerParams(use_tc_tiling_on_sc=False),
)
def _gather_sc(xnp_hbm, ii_hbm, out_hbm, idx_v, rows_v, sem):
    wid = lax.axis_index("s") * 2 + lax.axis_index("c")
    base = wid * _WPW

    def step(k, carry):
        w0 = base + k * _GU
        pltpu.sync_copy(ii_hbm.at[pl.ds(w0, _GU)], idx_v)
        cps = [pltpu.async_copy(xnp_hbm.at[idx_v.at[j]], rows_v.at[j], sem)
               for j in range(_GU)]
        for c in cps:
            c.wait()
        pltpu.sync_copy(rows_v, out_hbm.at[pl.ds(w0, _GU)])
        return carry

    lax.fori_loop(0, _WPW // _GU, step, 0)


def _silu(x):
    return x * jax.nn.sigmoid(x)


def _premix_body(cw_ref, xn_ref, attr_ref, msm_ref, msmT_ref, out_ref):
    xn = xn_ref[...]
    attr = attr_ref[...]
    bn = xn.shape[0]
    a = xn.reshape(bn * D, V)
    y = jnp.dot(a, msm_ref[...], preferred_element_type=jnp.float32)
    y = y.reshape(bn, D, V) * attr[:, None, :]
    z = jnp.dot(y.reshape(bn * D, V), msmT_ref[...],
                preferred_element_type=jnp.float32).reshape(bn, D, V)
    out_ref[...] = cw_ref[0] * xn + cw_ref[1] * z


def _premix(xn, xn_attr, Ms_M, cw):
    return pl.pallas_call(
        _premix_body,
        grid=(_NP // _BN,),
        in_specs=[
            pl.BlockSpec(memory_space=pltpu.SMEM),
            pl.BlockSpec((_BN, D, V), lambda i: (i, 0, 0)),
            pl.BlockSpec((_BN, V), lambda i: (i, 0)),
            pl.BlockSpec((V, V), lambda i: (0, 0)),
            pl.BlockSpec((V, V), lambda i: (0, 0)),
        ],
        out_specs=pl.BlockSpec((_BN, D, V), lambda i: (i, 0, 0)),
        out_shape=jax.ShapeDtypeStruct((_NP, D, V), jnp.float32),
    )(cw, xn, xn_attr, Ms_M, Ms_M.T)


def _edge_body(attr_ref, gsd_ref, f1_ref, b1_ref, f2_ref, b2_ref, w_ref,
               out_ref):
    a0 = attr_ref[:, 0:1]
    a1 = attr_ref[:, 1:2]
    w1 = jnp.concatenate(
        [_silu(a0 * f1_ref[...] + b1_ref[...]),
         _silu(a1 * f1_ref[...] + b1_ref[...])], axis=1)
    ws = gsd_ref[...] * w1
    mm = jnp.dot(ws, w_ref[...], preferred_element_type=jnp.float32)
    w2 = jnp.concatenate(
        [_silu(a0 * f2_ref[...] + b2_ref[...]),
         _silu(a1 * f2_ref[...] + b2_ref[...])], axis=1)
    out_ref[...] = mm * w2


def _edge(attr2, gsd, f1t, b1t, f2t, b2t, w192):
    return pl.pallas_call(
        _edge_body,
        grid=(_EP // 2 // _BE2,),
        in_specs=[
            pl.BlockSpec((_BE2, 2), lambda i: (i, 0)),
            pl.BlockSpec((_BE2, 192), lambda i: (i, 0)),
            pl.BlockSpec((1, 96), lambda i: (0, 0)),
            pl.BlockSpec((1, 96), lambda i: (0, 0)),
            pl.BlockSpec((1, 48), lambda i: (0, 0)),
            pl.BlockSpec((1, 48), lambda i: (0, 0)),
            pl.BlockSpec((192, 96), lambda i: (0, 0)),
        ],
        out_specs=pl.BlockSpec((_BE2, 96), lambda i: (i, 0)),
        out_shape=jax.ShapeDtypeStruct((_EP // 2, 96), jnp.float32),
    )(attr2, gsd, f1t, b1t, f2t, b2t, w192)


def _final_body(a1_ref, a2_ref, cd_ref, out_ref):
    a1 = a1_ref[...]
    a2 = a2_ref[...]
    bn = a1.shape[0]
    t = (jnp.dot(a1.reshape(bn * D, V), cd_ref[0:V, :],
                 preferred_element_type=jnp.float32)
         + jnp.dot(a2.reshape(bn * D, V), cd_ref[V:2 * V, :],
                   preferred_element_type=jnp.float32)).reshape(bn, D, V)
    nrm = jnp.sqrt(jnp.sum(t * t, axis=1))
    out_ref[...] = t * jnp.tanh(nrm)[:, None, :]


_BNF = 2000


def _final(acc1, acc2, cd):
    return pl.pallas_call(
        _final_body,
        grid=(N // _BNF,),
        in_specs=[
            pl.BlockSpec((_BNF, D, V), lambda i: (i, 0, 0)),
            pl.BlockSpec((_BNF, D, V), lambda i: (i, 0, 0)),
            pl.BlockSpec((2 * V, V), lambda i: (0, 0)),
        ],
        out_specs=pl.BlockSpec((_BNF, D, V), lambda i: (i, 0, 0)),
        out_shape=jax.ShapeDtypeStruct((N, D, V), jnp.float32),
    )(acc1, acc2, cd)


def kernel(xn, xn_attr, xe_attr, xe_src, xe_dst, Ms_M, Ms_w, fc1_w, fc1_b,
           fc2_w, fc2_b, n2e_M1, n2e_M2, mvv_M1, mvv_M2, e2n_M1, e2n_M2):
    ang = 0.1 * Ms_w[0]
    cw = jnp.stack([jnp.cos(ang), jnp.sin(ang)])
    Rm = (mvv_M1 + mvv_M2) * 0.5
    A = (n2e_M1 * 0.5 + n2e_M2 * 0.25) @ Rm
    B = (-n2e_M1 * 0.5 + n2e_M2 * 0.25) @ Rm
    eye3 = jnp.eye(3, dtype=jnp.float32)
    bda = jnp.kron(eye3, A)
    bdb = jnp.kron(eye3, B)
    w96 = jnp.concatenate([bda, bdb], axis=0)
    w192 = jnp.kron(jnp.eye(2, dtype=jnp.float32), w96)
    cd = jnp.concatenate([(e2n_M1 + e2n_M2) * 0.5,
                          (e2n_M2 - e2n_M1) * 0.5], axis=0)
    f1t = jnp.tile(fc1_w[:, 0], 6)[None, :]
    b1t = jnp.tile(fc1_b, 6)[None, :]
    f2t = jnp.tile(fc2_w[:, 0], 3)[None, :]
    b2t = jnp.tile(fc2_b, 3)[None, :]

    xn_p = jnp.concatenate(
        [xn, jnp.zeros((_NP - N, D, V), jnp.float32)], axis=0)
    attr_np = jnp.concatenate(
        [xn_attr, jnp.zeros((_NP - N, V), jnp.float32)], axis=0)
    xnp = _premix(xn_p, attr_np, Ms_M, cw).reshape(_NP, D * V)
    xnp_sc = _relayout_sc(xnp)

    ii = jnp.stack([xe_src, xe_dst], axis=1).reshape(2 * E)
    pad = jnp.arange(2 * _EP - 2 * E, dtype=jnp.int32) % N
    ii3 = jnp.concatenate([ii, pad]).reshape(_IW, 128)
    gsd = _gather_sc(xnp_sc, ii3).reshape(_EP // 2, 4 * D * V)

    attr_p = jnp.concatenate(
        [xe_attr, jnp.zeros((_EP - E,), jnp.float32)]).reshape(_EP // 2, 2)
    m2 = _edge(attr_p, gsd, f1t, b1t, f2t, b2t, w192)

    m = m2.reshape(_EP, D, V)[:E]
    acc1 = jax.ops.segment_sum(m, xe_dst, num_segments=N)
    acc2 = jax.ops.segment_sum(m, xe_src, num_segments=N)

    return _final(acc1, acc2, cd)


# speedup vs baseline: 42.4598x; 39.0948x over previous
import functools

import jax
import jax.numpy as jnp
from jax import lax
from jax.experimental import pallas as pl
from jax.experimental.pallas import tpu as pltpu
from jax.experimental.pallas import tpu_sc as plsc

N = 100000
E = 1600000
D = 3
V = 16

_BN = 3128
_BE2 = 6400

_EP = 1638400
_IW = 2 * _EP // 128
_WPW = _IW // 32
_GU = 8

_mesh = plsc.VectorSubcoreMesh(core_axis_name="c", subcore_axis_name="s")

_NP = 100096


@functools.partial(
    pl.kernel,
    out_type=jax.ShapeDtypeStruct((_NP, 48), jnp.float32),
    mesh=_mesh,
    scratch_types=[pltpu.VMEM((184, 48), jnp.float32)],
    compiler_params=pltpu.CompilerParams(use_tc_tiling_on_sc=False),
)
def _relayout_sc(src_hbm, out_hbm, buf):
    wid = lax.axis_index("s") * 2 + lax.axis_index("c")
    base = wid * 3128

    def step(k, carry):
        r0 = base + k * 184
        pltpu.sync_copy(src_hbm.at[pl.ds(r0, 184)], buf)
        pltpu.sync_copy(buf, out_hbm.at[pl.ds(r0, 184)])
        return carry

    lax.fori_loop(0, 17, step, 0)


@functools.partial(
    pl.kernel,
    out_type=jax.ShapeDtypeStruct((_IW, 128, 48), jnp.float32),
    mesh=_mesh,
    scratch_types=[
        pltpu.VMEM((_GU, 128), jnp.int32),
        pltpu.VMEM((_GU, 128, 48), jnp.float32),
        pltpu.SemaphoreType.DMA,
    ],
    compiler_params=pltpu.CompilerParams(use_tc_tiling_on_sc=False),
)
def _gather_sc(xnp_hbm, ii_hbm, out_hbm, idx_v, rows_v, sem):
    wid = lax.axis_index("s") * 2 + lax.axis_index("c")
    base = wid * _WPW

    def step(k, carry):
        w0 = base + k * _GU
        pltpu.sync_copy(ii_hbm.at[pl.ds(w0, _GU)], idx_v)
        cps = [pltpu.async_copy(xnp_hbm.at[idx_v.at[j]], rows_v.at[j], sem)
               for j in range(_GU)]
        for c in cps:
            c.wait()
        pltpu.sync_copy(rows_v, out_hbm.at[pl.ds(w0, _GU)])
        return carry

    lax.fori_loop(0, _WPW // _GU, step, 0)


def _silu(x):
    return x * jax.nn.sigmoid(x)


def _premix_body(cw_ref, xn_ref, attr_ref, msm_ref, msmT_ref, out_ref):
    xn = xn_ref[...]
    attr = attr_ref[...]
    bn = xn.shape[0]
    a = xn.reshape(bn * D, V)
    y = jnp.dot(a, msm_ref[...], preferred_element_type=jnp.float32)
    y = y.reshape(bn, D, V) * attr[:, None, :]
    z = jnp.dot(y.reshape(bn * D, V), msmT_ref[...],
                preferred_element_type=jnp.float32).reshape(bn, D, V)
    out_ref[...] = cw_ref[0] * xn + cw_ref[1] * z


def _premix(xn, xn_attr, Ms_M, cw):
    return pl.pallas_call(
        _premix_body,
        grid=(_NP // _BN,),
        in_specs=[
            pl.BlockSpec(memory_space=pltpu.SMEM),
            pl.BlockSpec((_BN, D, V), lambda i: (i, 0, 0)),
            pl.BlockSpec((_BN, V), lambda i: (i, 0)),
            pl.BlockSpec((V, V), lambda i: (0, 0)),
            pl.BlockSpec((V, V), lambda i: (0, 0)),
        ],
        out_specs=pl.BlockSpec((_BN, D, V), lambda i: (i, 0, 0)),
        out_shape=jax.ShapeDtypeStruct((_NP, D, V), jnp.float32),
    )(cw, xn, xn_attr, Ms_M, Ms_M.T)


_NR = 33408
_NA = 100224
_SW = _EP // 128
_TRASH = 1024


@functools.partial(
    pl.kernel,
    out_type=(jax.ShapeDtypeStruct((_NA, 48), jnp.float32),
              jax.ShapeDtypeStruct((_NA, 48), jnp.float32)),
    mesh=_mesh,
    scratch_types=[
        pltpu.VMEM_SHARED((_NR + _TRASH, 48), jnp.float32),
        pltpu.VMEM((4, 128), jnp.int32),
        pltpu.VMEM((4, 128, 48), jnp.float32),
        pltpu.VMEM((4, 128), jnp.int32),
        pltpu.SemaphoreType.DMA,
    ],
    compiler_params=pltpu.CompilerParams(use_tc_tiling_on_sc=False),
)
def _scatter_sc(m3, dst3, src3, zeros_hbm, acc1_hbm, acc2_hbm,
                spacc, dv, mv, tix, sem):
    cid = lax.axis_index("c")
    sid = lax.axis_index("s")
    zrows = (_NR + _TRASH) // 16
    frows = _NR // 16

    def run(idx3, acc_hbm):
        for r in range(3):
            rbase = r * _NR
            pltpu.sync_copy(zeros_hbm, spacc.at[pl.ds(sid * zrows, zrows)])
            plsc.subcore_barrier()

            def step(k, carry):
                w0 = sid * 800 + k * 4
                pltpu.sync_copy(idx3.at[pl.ds(w0, 4)], dv)
                pltpu.sync_copy(m3.at[pl.ds(w0, 4)], mv)
                lane = lax.iota(jnp.int32, 16)
                for j in range(4):
                    for c in range(8):
                        t = dv[j, pl.ds(c * 16, 16)] - rbase
                        ok = (t >= 0) & (t < _NR)
                        tr = _NR + lane + 16 * ((j * 8 + c) % 64)
                        tix[j, pl.ds(c * 16, 16)] = jnp.where(ok, t, tr)
                cps = [pltpu.async_copy(mv.at[j], spacc.at[tix.at[j]], sem,
                                        add=True) for j in range(4)]
                for cp in cps:
                    cp.wait()
                return carry

            lax.fori_loop(0, 200, step, 0)
            plsc.subcore_barrier()
            pltpu.sync_copy(spacc.at[pl.ds(sid * frows, frows)],
                            acc_hbm.at[pl.ds(rbase + sid * frows, frows)])
            plsc.subcore_barrier()

    @pl.when(cid == 0)
    def _():
        run(dst3, acc1_hbm)

    @pl.when(cid == 1)
    def _():
        run(src3, acc2_hbm)


def _edge_body(attr_ref, gsd_ref, f1_ref, b1_ref, f2_ref, b2_ref, w_ref,
               out_ref):
    a0 = attr_ref[:, 0:1]
    a1 = attr_ref[:, 1:2]
    w1 = jnp.concatenate(
        [_silu(a0 * f1_ref[...] + b1_ref[...]),
         _silu(a1 * f1_ref[...] + b1_ref[...])], axis=1)
    ws = gsd_ref[...] * w1
    mm = jnp.dot(ws, w_ref[...], preferred_element_type=jnp.float32)
    w2 = jnp.concatenate(
        [_silu(a0 * f2_ref[...] + b2_ref[...]),
         _silu(a1 * f2_ref[...] + b2_ref[...])], axis=1)
    out_ref[...] = mm * w2


def _edge(attr2, gsd, f1t, b1t, f2t, b2t, w192):
    return pl.pallas_call(
        _edge_body,
        grid=(_EP // 2 // _BE2,),
        in_specs=[
            pl.BlockSpec((_BE2, 2), lambda i: (i, 0)),
            pl.BlockSpec((_BE2, 192), lambda i: (i, 0)),
            pl.BlockSpec((1, 96), lambda i: (0, 0)),
            pl.BlockSpec((1, 96), lambda i: (0, 0)),
            pl.BlockSpec((1, 48), lambda i: (0, 0)),
            pl.BlockSpec((1, 48), lambda i: (0, 0)),
            pl.BlockSpec((192, 96), lambda i: (0, 0)),
        ],
        out_specs=pl.BlockSpec((_BE2, 96), lambda i: (i, 0)),
        out_shape=jax.ShapeDtypeStruct((_EP // 2, 96), jnp.float32),
    )(attr2, gsd, f1t, b1t, f2t, b2t, w192)


def _final_body(a1_ref, a2_ref, cd_ref, out_ref):
    a1 = a1_ref[...]
    a2 = a2_ref[...]
    bn = a1.shape[0]
    t = (jnp.dot(a1.reshape(bn * D, V), cd_ref[0:V, :],
                 preferred_element_type=jnp.float32)
         + jnp.dot(a2.reshape(bn * D, V), cd_ref[V:2 * V, :],
                   preferred_element_type=jnp.float32)).reshape(bn, D, V)
    nrm = jnp.sqrt(jnp.sum(t * t, axis=1))
    out_ref[...] = t * jnp.tanh(nrm)[:, None, :]


_BNF = 2000


def _final(acc1, acc2, cd):
    return pl.pallas_call(
        _final_body,
        grid=(N // _BNF,),
        in_specs=[
            pl.BlockSpec((_BNF, D, V), lambda i: (i, 0, 0)),
            pl.BlockSpec((_BNF, D, V), lambda i: (i, 0, 0)),
            pl.BlockSpec((2 * V, V), lambda i: (0, 0)),
        ],
        out_specs=pl.BlockSpec((_BNF, D, V), lambda i: (i, 0, 0)),
        out_shape=jax.ShapeDtypeStruct((N, D, V), jnp.float32),
    )(acc1, acc2, cd)


def kernel(xn, xn_attr, xe_attr, xe_src, xe_dst, Ms_M, Ms_w, fc1_w, fc1_b,
           fc2_w, fc2_b, n2e_M1, n2e_M2, mvv_M1, mvv_M2, e2n_M1, e2n_M2):
    ang = 0.1 * Ms_w[0]
    cw = jnp.stack([jnp.cos(ang), jnp.sin(ang)])
    Rm = (mvv_M1 + mvv_M2) * 0.5
    A = (n2e_M1 * 0.5 + n2e_M2 * 0.25) @ Rm
    B = (-n2e_M1 * 0.5 + n2e_M2 * 0.25) @ Rm
    eye3 = jnp.eye(3, dtype=jnp.float32)
    bda = jnp.kron(eye3, A)
    bdb = jnp.kron(eye3, B)
    w96 = jnp.concatenate([bda, bdb], axis=0)
    w192 = jnp.kron(jnp.eye(2, dtype=jnp.float32), w96)
    cd = jnp.concatenate([(e2n_M1 + e2n_M2) * 0.5,
                          (e2n_M2 - e2n_M1) * 0.5], axis=0)
    f1t = jnp.tile(fc1_w[:, 0], 6)[None, :]
    b1t = jnp.tile(fc1_b, 6)[None, :]
    f2t = jnp.tile(fc2_w[:, 0], 3)[None, :]
    b2t = jnp.tile(fc2_b, 3)[None, :]

    xn_p = jnp.concatenate(
        [xn, jnp.zeros((_NP - N, D, V), jnp.float32)], axis=0)
    attr_np = jnp.concatenate(
        [xn_attr, jnp.zeros((_NP - N, V), jnp.float32)], axis=0)
    xnp = _premix(xn_p, attr_np, Ms_M, cw).reshape(_NP, D * V)
    xnp_sc = _relayout_sc(xnp)

    ii = jnp.stack([xe_src, xe_dst], axis=1).reshape(2 * E)
    pad = jnp.arange(2 * _EP - 2 * E, dtype=jnp.int32) % N
    ii3 = jnp.concatenate([ii, pad]).reshape(_IW, 128)
    gsd = _gather_sc(xnp_sc, ii3).reshape(_EP // 2, 4 * D * V)

    attr_p = jnp.concatenate(
        [xe_attr, jnp.zeros((_EP - E,), jnp.float32)]).reshape(_EP // 2, 2)
    m2 = _edge(attr_p, gsd, f1t, b1t, f2t, b2t, w192)

    ipad = jnp.full((_EP - E,), 1000000, jnp.int32)
    dst3 = jnp.concatenate([xe_dst, ipad]).reshape(_SW, 128)
    src3 = jnp.concatenate([xe_src, ipad]).reshape(_SW, 128)
    m3 = m2.reshape(_SW, 128, 48)
    zeros = jnp.zeros(((_NR + _TRASH) // 16, 48), jnp.float32)
    acc1p, acc2p = _scatter_sc(m3, dst3, src3, zeros)

    acc1 = acc1p[:N].reshape(N, D, V)
    acc2 = acc2p[:N].reshape(N, D, V)
    return _final(acc1, acc2, cd)


# speedup vs baseline: 44.6904x; 1.0525x over previous
import functools

import jax
import jax.numpy as jnp
from jax import lax
from jax.experimental import pallas as pl
from jax.experimental.pallas import tpu as pltpu
from jax.experimental.pallas import tpu_sc as plsc

N = 100000
E = 1600000
D = 3
V = 16

_BN = 3128
_BE2 = 6400

_EP = 1638400
_IW = 2 * _EP // 128
_WPW = _IW // 32
_GU = 8

_mesh = plsc.VectorSubcoreMesh(core_axis_name="c", subcore_axis_name="s")

_NP = 100096


@functools.partial(
    pl.kernel,
    out_type=jax.ShapeDtypeStruct((_NP, 48), jnp.float32),
    mesh=_mesh,
    scratch_types=[pltpu.VMEM((184, 48), jnp.float32)],
    compiler_params=pltpu.CompilerParams(use_tc_tiling_on_sc=False),
)
def _relayout_sc(src_hbm, out_hbm, buf):
    wid = lax.axis_index("s") * 2 + lax.axis_index("c")
    base = wid * 3128

    def step(k, carry):
        r0 = base + k * 184
        pltpu.sync_copy(src_hbm.at[pl.ds(r0, 184)], buf)
        pltpu.sync_copy(buf, out_hbm.at[pl.ds(r0, 184)])
        return carry

    lax.fori_loop(0, 17, step, 0)


@functools.partial(
    pl.kernel,
    out_type=jax.ShapeDtypeStruct((_IW, 128, 48), jnp.float32),
    mesh=_mesh,
    scratch_types=[
        pltpu.VMEM((_GU, 128), jnp.int32),
        pltpu.VMEM((_GU, 128, 48), jnp.float32),
        pltpu.SemaphoreType.DMA,
    ],
    compiler_params=pltpu.CompilerParams(use_tc_tiling_on_sc=False),
)
def _gather_sc(xnp_hbm, ii_hbm, out_hbm, idx_v, rows_v, sem):
    wid = lax.axis_index("s") * 2 + lax.axis_index("c")
    base = wid * _WPW

    def step(k, carry):
        w0 = base + k * _GU
        pltpu.sync_copy(ii_hbm.at[pl.ds(w0, _GU)], idx_v)
        cps = [pltpu.async_copy(xnp_hbm.at[idx_v.at[j]], rows_v.at[j], sem)
               for j in range(_GU)]
        for c in cps:
            c.wait()
        pltpu.sync_copy(rows_v, out_hbm.at[pl.ds(w0, _GU)])
        return carry

    lax.fori_loop(0, _WPW // _GU, step, 0)


def _silu(x):
    return x * jax.nn.sigmoid(x)


def _premix_body(cw_ref, xn_ref, attr_ref, msm_ref, msmT_ref, out_ref):
    xn = xn_ref[...]
    attr = attr_ref[...]
    bn = xn.shape[0]
    a = xn.reshape(bn * D, V)
    y = jnp.dot(a, msm_ref[...], preferred_element_type=jnp.float32)
    y = y.reshape(bn, D, V) * attr[:, None, :]
    z = jnp.dot(y.reshape(bn * D, V), msmT_ref[...],
                preferred_element_type=jnp.float32).reshape(bn, D, V)
    out_ref[...] = cw_ref[0] * xn + cw_ref[1] * z


def _premix(xn, xn_attr, Ms_M, cw):
    return pl.pallas_call(
        _premix_body,
        grid=(_NP // _BN,),
        in_specs=[
            pl.BlockSpec(memory_space=pltpu.SMEM),
            pl.BlockSpec((_BN, D, V), lambda i: (i, 0, 0)),
            pl.BlockSpec((_BN, V), lambda i: (i, 0)),
            pl.BlockSpec((V, V), lambda i: (0, 0)),
            pl.BlockSpec((V, V), lambda i: (0, 0)),
        ],
        out_specs=pl.BlockSpec((_BN, D, V), lambda i: (i, 0, 0)),
        out_shape=jax.ShapeDtypeStruct((_NP, D, V), jnp.float32),
    )(cw, xn, xn_attr, Ms_M, Ms_M.T)


_NR = 33408
_NA = 100224
_SW = _EP // 128
_TRASH = 1024


@functools.partial(
    pl.kernel,
    out_type=(jax.ShapeDtypeStruct((_NA, 48), jnp.float32),
              jax.ShapeDtypeStruct((_NA, 48), jnp.float32)),
    mesh=_mesh,
    scratch_types=[
        pltpu.VMEM_SHARED((_NR + _TRASH, 48), jnp.float32),
        pltpu.VMEM((2, 2, 128), jnp.int32),
        pltpu.VMEM((2, 2, 128, 48), jnp.float32),
        pltpu.VMEM((2, 128), jnp.int32),
        pltpu.SemaphoreType.DMA,
        pltpu.SemaphoreType.DMA,
    ],
    compiler_params=pltpu.CompilerParams(use_tc_tiling_on_sc=False),
)
def _scatter_sc(m3, dst3, src3, zeros_hbm, acc1_hbm, acc2_hbm,
                spacc, dv, mv, tix, sem_in, sem_add):
    cid = lax.axis_index("c")
    sid = lax.axis_index("s")
    zrows = (_NR + _TRASH) // 16
    frows = _NR // 16

    nit = 400
    u2 = 2

    def run(idx3, acc_hbm):
        for r in range(3):
            rbase = r * _NR
            pltpu.sync_copy(zeros_hbm, spacc.at[pl.ds(sid * zrows, zrows)])
            plsc.subcore_barrier()

            def fetch(k, p):
                w0 = sid * 800 + k * u2
                pltpu.async_copy(idx3.at[pl.ds(w0, u2)], dv.at[p], sem_in)
                pltpu.async_copy(m3.at[pl.ds(w0, u2)], mv.at[p], sem_in)

            def drain_in(p):
                pltpu.make_async_copy(idx3.at[pl.ds(0, u2)], dv.at[p],
                                      sem_in).wait()
                pltpu.make_async_copy(m3.at[pl.ds(0, u2)], mv.at[p],
                                      sem_in).wait()

            fetch(0, 0)

            def step(k, carry):
                p = lax.rem(k, 2)
                drain_in(p)

                @pl.when(k < nit - 1)
                def _():
                    fetch(k + 1, 1 - p)

                lane = lax.iota(jnp.int32, 16)
                for j in range(u2):
                    for c in range(8):
                        t = dv[p, j, pl.ds(c * 16, 16)] - rbase
                        ok = (t >= 0) & (t < _NR)
                        tr = _NR + lane + 16 * ((j * 8 + c) % 64)
                        tix[j, pl.ds(c * 16, 16)] = jnp.where(ok, t, tr)
                cps = [pltpu.async_copy(mv.at[p, j], spacc.at[tix.at[j]],
                                        sem_add, add=True)
                       for j in range(u2)]
                for cp in cps:
                    cp.wait()
                return carry

            lax.fori_loop(0, nit, step, 0)
            plsc.subcore_barrier()
            pltpu.sync_copy(spacc.at[pl.ds(sid * frows, frows)],
                            acc_hbm.at[pl.ds(rbase + sid * frows, frows)])
            plsc.subcore_barrier()

    @pl.when(cid == 0)
    def _():
        run(dst3, acc1_hbm)

    @pl.when(cid == 1)
    def _():
        run(src3, acc2_hbm)


def _edge_body(attr_ref, gsd_ref, f1_ref, b1_ref, f2_ref, b2_ref, w_ref,
               out_ref):
    a0 = attr_ref[:, 0:1]
    a1 = attr_ref[:, 1:2]
    w1 = jnp.concatenate(
        [_silu(a0 * f1_ref[...] + b1_ref[...]),
         _silu(a1 * f1_ref[...] + b1_ref[...])], axis=1)
    ws = gsd_ref[...] * w1
    mm = jnp.dot(ws, w_ref[...], preferred_element_type=jnp.float32)
    w2 = jnp.concatenate(
        [_silu(a0 * f2_ref[...] + b2_ref[...]),
         _silu(a1 * f2_ref[...] + b2_ref[...])], axis=1)
    out_ref[...] = mm * w2


def _edge(attr2, gsd, f1t, b1t, f2t, b2t, w192):
    return pl.pallas_call(
        _edge_body,
        grid=(_EP // 2 // _BE2,),
        in_specs=[
            pl.BlockSpec((_BE2, 2), lambda i: (i, 0)),
            pl.BlockSpec((_BE2, 192), lambda i: (i, 0)),
            pl.BlockSpec((1, 96), lambda i: (0, 0)),
            pl.BlockSpec((1, 96), lambda i: (0, 0)),
            pl.BlockSpec((1, 48), lambda i: (0, 0)),
            pl.BlockSpec((1, 48), lambda i: (0, 0)),
            pl.BlockSpec((192, 96), lambda i: (0, 0)),
        ],
        out_specs=pl.BlockSpec((_BE2, 96), lambda i: (i, 0)),
        out_shape=jax.ShapeDtypeStruct((_EP // 2, 96), jnp.float32),
    )(attr2, gsd, f1t, b1t, f2t, b2t, w192)


def _final_body(a1_ref, a2_ref, cd_ref, out_ref):
    a1 = a1_ref[...]
    a2 = a2_ref[...]
    bn = a1.shape[0]
    t = (jnp.dot(a1.reshape(bn * D, V), cd_ref[0:V, :],
                 preferred_element_type=jnp.float32)
         + jnp.dot(a2.reshape(bn * D, V), cd_ref[V:2 * V, :],
                   preferred_element_type=jnp.float32)).reshape(bn, D, V)
    nrm = jnp.sqrt(jnp.sum(t * t, axis=1))
    out_ref[...] = t * jnp.tanh(nrm)[:, None, :]


_BNF = 2000


def _final(acc1, acc2, cd):
    return pl.pallas_call(
        _final_body,
        grid=(N // _BNF,),
        in_specs=[
            pl.BlockSpec((_BNF, D, V), lambda i: (i, 0, 0)),
            pl.BlockSpec((_BNF, D, V), lambda i: (i, 0, 0)),
            pl.BlockSpec((2 * V, V), lambda i: (0, 0)),
        ],
        out_specs=pl.BlockSpec((_BNF, D, V), lambda i: (i, 0, 0)),
        out_shape=jax.ShapeDtypeStruct((N, D, V), jnp.float32),
    )(acc1, acc2, cd)


def kernel(xn, xn_attr, xe_attr, xe_src, xe_dst, Ms_M, Ms_w, fc1_w, fc1_b,
           fc2_w, fc2_b, n2e_M1, n2e_M2, mvv_M1, mvv_M2, e2n_M1, e2n_M2):
    ang = 0.1 * Ms_w[0]
    cw = jnp.stack([jnp.cos(ang), jnp.sin(ang)])
    Rm = (mvv_M1 + mvv_M2) * 0.5
    A = (n2e_M1 * 0.5 + n2e_M2 * 0.25) @ Rm
    B = (-n2e_M1 * 0.5 + n2e_M2 * 0.25) @ Rm
    eye3 = jnp.eye(3, dtype=jnp.float32)
    bda = jnp.kron(eye3, A)
    bdb = jnp.kron(eye3, B)
    w96 = jnp.concatenate([bda, bdb], axis=0)
    w192 = jnp.kron(jnp.eye(2, dtype=jnp.float32), w96)
    cd = jnp.concatenate([(e2n_M1 + e2n_M2) * 0.5,
                          (e2n_M2 - e2n_M1) * 0.5], axis=0)
    f1t = jnp.tile(fc1_w[:, 0], 6)[None, :]
    b1t = jnp.tile(fc1_b, 6)[None, :]
    f2t = jnp.tile(fc2_w[:, 0], 3)[None, :]
    b2t = jnp.tile(fc2_b, 3)[None, :]

    xn_p = jnp.concatenate(
        [xn, jnp.zeros((_NP - N, D, V), jnp.float32)], axis=0)
    attr_np = jnp.concatenate(
        [xn_attr, jnp.zeros((_NP - N, V), jnp.float32)], axis=0)
    xnp = _premix(xn_p, attr_np, Ms_M, cw).reshape(_NP, D * V)
    xnp_sc = _relayout_sc(xnp)

    ii = jnp.stack([xe_src, xe_dst], axis=1).reshape(2 * E)
    pad = jnp.arange(2 * _EP - 2 * E, dtype=jnp.int32) % N
    ii3 = jnp.concatenate([ii, pad]).reshape(_IW, 128)
    gsd = _gather_sc(xnp_sc, ii3).reshape(_EP // 2, 4 * D * V)

    attr_p = jnp.concatenate(
        [xe_attr, jnp.zeros((_EP - E,), jnp.float32)]).reshape(_EP // 2, 2)
    m2 = _edge(attr_p, gsd, f1t, b1t, f2t, b2t, w192)

    ipad = jnp.full((_EP - E,), 1000000, jnp.int32)
    dst3 = jnp.concatenate([xe_dst, ipad]).reshape(_SW, 128)
    src3 = jnp.concatenate([xe_src, ipad]).reshape(_SW, 128)
    m3 = m2.reshape(_SW, 128, 48)
    zeros = jnp.zeros(((_NR + _TRASH) // 16, 48), jnp.float32)
    acc1p, acc2p = _scatter_sc(m3, dst3, src3, zeros)

    acc1 = acc1p[:N].reshape(N, D, V)
    acc2 = acc2p[:N].reshape(N, D, V)
    return _final(acc1, acc2, cd)


# speedup vs baseline: 44.9270x; 1.0053x over previous
import functools

import jax
import jax.numpy as jnp
from jax import lax
from jax.experimental import pallas as pl
from jax.experimental.pallas import tpu as pltpu
from jax.experimental.pallas import tpu_sc as plsc

N = 100000
E = 1600000
D = 3
V = 16

_BN = 3128
_BE2 = 6400

_EP = 1638400
_IW = 2 * _EP // 128
_WPW = _IW // 32
_GU = 8

_mesh = plsc.VectorSubcoreMesh(core_axis_name="c", subcore_axis_name="s")

_NP = 100096


@functools.partial(
    pl.kernel,
    out_type=jax.ShapeDtypeStruct((_NP, 48), jnp.float32),
    mesh=_mesh,
    scratch_types=[pltpu.VMEM((184, 48), jnp.float32)],
    compiler_params=pltpu.CompilerParams(use_tc_tiling_on_sc=False),
)
def _relayout_sc(src_hbm, out_hbm, buf):
    wid = lax.axis_index("s") * 2 + lax.axis_index("c")
    base = wid * 3128

    def step(k, carry):
        r0 = base + k * 184
        pltpu.sync_copy(src_hbm.at[pl.ds(r0, 184)], buf)
        pltpu.sync_copy(buf, out_hbm.at[pl.ds(r0, 184)])
        return carry

    lax.fori_loop(0, 17, step, 0)


@functools.partial(
    pl.kernel,
    out_type=jax.ShapeDtypeStruct((_IW, 128, 48), jnp.float32),
    mesh=_mesh,
    scratch_types=[
        pltpu.VMEM((2, _GU, 128), jnp.int32),
        pltpu.VMEM((2, _GU, 128, 48), jnp.float32),
        pltpu.SemaphoreType.DMA,
        pltpu.SemaphoreType.DMA,
    ],
    compiler_params=pltpu.CompilerParams(use_tc_tiling_on_sc=False),
)
def _gather_sc(xnp_hbm, ii_hbm, out_hbm, idx_v, rows_v, sem_g, sem_out):
    wid = lax.axis_index("s") * 2 + lax.axis_index("c")
    base = wid * _WPW
    nit = _WPW // _GU

    def fire(k, p):
        w0 = base + k * _GU
        pltpu.sync_copy(ii_hbm.at[pl.ds(w0, _GU)], idx_v.at[p])
        for j in range(_GU):
            pltpu.async_copy(xnp_hbm.at[idx_v.at[p, j]], rows_v.at[p, j],
                             sem_g)

    def wait_rows(p):
        pltpu.make_async_copy(out_hbm.at[pl.ds(0, _GU)], rows_v.at[p],
                              sem_g).wait()

    def wait_out(p):
        pltpu.make_async_copy(out_hbm.at[pl.ds(0, _GU)], rows_v.at[p],
                              sem_out).wait()

    fire(0, 0)

    def step(k, carry):
        p = lax.rem(k, 2)
        q = 1 - p
        wait_rows(p)

        @pl.when(k > 0)
        def _():
            wait_out(q)

        @pl.when(k < nit - 1)
        def _():
            fire(k + 1, q)

        w0 = base + k * _GU
        pltpu.async_copy(rows_v.at[p], out_hbm.at[pl.ds(w0, _GU)], sem_out)
        return carry

    lax.fori_loop(0, nit, step, 0)
    wait_out((nit - 1) % 2)


def _silu(x):
    return x * jax.nn.sigmoid(x)


def _premix_body(cw_ref, xn_ref, attr_ref, msm_ref, msmT_ref, out_ref):
    xn = xn_ref[...]
    attr = attr_ref[...]
    bn = xn.shape[0]
    a = xn.reshape(bn * D, V)
    y = jnp.dot(a, msm_ref[...], preferred_element_type=jnp.float32)
    y = y.reshape(bn, D, V) * attr[:, None, :]
    z = jnp.dot(y.reshape(bn * D, V), msmT_ref[...],
                preferred_element_type=jnp.float32).reshape(bn, D, V)
    out_ref[...] = cw_ref[0] * xn + cw_ref[1] * z


def _premix(xn, xn_attr, Ms_M, cw):
    return pl.pallas_call(
        _premix_body,
        grid=(_NP // _BN,),
        in_specs=[
            pl.BlockSpec(memory_space=pltpu.SMEM),
            pl.BlockSpec((_BN, D, V), lambda i: (i, 0, 0)),
            pl.BlockSpec((_BN, V), lambda i: (i, 0)),
            pl.BlockSpec((V, V), lambda i: (0, 0)),
            pl.BlockSpec((V, V), lambda i: (0, 0)),
        ],
        out_specs=pl.BlockSpec((_BN, D, V), lambda i: (i, 0, 0)),
        out_shape=jax.ShapeDtypeStruct((_NP, D, V), jnp.float32),
    )(cw, xn, xn_attr, Ms_M, Ms_M.T)


_NR = 33408
_NA = 100224
_SW = _EP // 128
_TRASH = 1024


@functools.partial(
    pl.kernel,
    out_type=(jax.ShapeDtypeStruct((_NA, 48), jnp.float32),
              jax.ShapeDtypeStruct((_NA, 48), jnp.float32)),
    mesh=_mesh,
    scratch_types=[
        pltpu.VMEM_SHARED((_NR + _TRASH, 48), jnp.float32),
        pltpu.VMEM((2, 2, 128), jnp.int32),
        pltpu.VMEM((2, 2, 128, 48), jnp.float32),
        pltpu.VMEM((2, 2, 128), jnp.int32),
        pltpu.SemaphoreType.DMA,
        pltpu.SemaphoreType.DMA,
    ],
    compiler_params=pltpu.CompilerParams(use_tc_tiling_on_sc=False),
)
def _scatter_sc(m3, dst3, src3, zeros_hbm, acc1_hbm, acc2_hbm,
                spacc, dv, mv, tix, sem_in, sem_add):
    cid = lax.axis_index("c")
    sid = lax.axis_index("s")
    zrows = (_NR + _TRASH) // 16
    frows = _NR // 16

    nit = 400
    u2 = 2

    def run(idx3, acc_hbm):
        for r in range(3):
            rbase = r * _NR
            pltpu.sync_copy(zeros_hbm, spacc.at[pl.ds(sid * zrows, zrows)])
            plsc.subcore_barrier()

            def fetch(k, p):
                w0 = sid * 800 + k * u2
                pltpu.async_copy(idx3.at[pl.ds(w0, u2)], dv.at[p], sem_in)
                pltpu.async_copy(m3.at[pl.ds(w0, u2)], mv.at[p], sem_in)

            def drain_in(p):
                pltpu.make_async_copy(idx3.at[pl.ds(0, u2)], dv.at[p],
                                      sem_in).wait()
                pltpu.make_async_copy(m3.at[pl.ds(0, u2)], mv.at[p],
                                      sem_in).wait()

            def wait_adds(p):
                pltpu.make_async_copy(m3.at[pl.ds(0, u2)], mv.at[p],
                                      sem_add).wait()

            fetch(0, 0)

            def step(k, carry):
                p = lax.rem(k, 2)
                drain_in(p)

                @pl.when(k > 0)
                def _():
                    wait_adds(1 - p)

                @pl.when(k < nit - 1)
                def _():
                    fetch(k + 1, 1 - p)

                lane = lax.iota(jnp.int32, 16)
                for j in range(u2):
                    for c in range(8):
                        t = dv[p, j, pl.ds(c * 16, 16)] - rbase
                        ok = (t >= 0) & (t < _NR)
                        tr = _NR + lane + 16 * ((j * 8 + c) % 64)
                        tix[p, j, pl.ds(c * 16, 16)] = jnp.where(ok, t, tr)
                for j in range(u2):
                    pltpu.async_copy(mv.at[p, j], spacc.at[tix.at[p, j]],
                                     sem_add, add=True)
                return carry

            lax.fori_loop(0, nit, step, 0)
            wait_adds((nit - 1) % 2)
            plsc.subcore_barrier()
            pltpu.sync_copy(spacc.at[pl.ds(sid * frows, frows)],
                            acc_hbm.at[pl.ds(rbase + sid * frows, frows)])
            plsc.subcore_barrier()

    @pl.when(cid == 0)
    def _():
        run(dst3, acc1_hbm)

    @pl.when(cid == 1)
    def _():
        run(src3, acc2_hbm)


def _edge_body(attr_ref, gsd_ref, f1_ref, b1_ref, f2_ref, b2_ref, w_ref,
               out_ref):
    a0 = attr_ref[:, 0:1]
    a1 = attr_ref[:, 1:2]
    w1 = jnp.concatenate(
        [_silu(a0 * f1_ref[...] + b1_ref[...]),
         _silu(a1 * f1_ref[...] + b1_ref[...])], axis=1)
    ws = gsd_ref[...] * w1
    mm = jnp.dot(ws, w_ref[...], preferred_element_type=jnp.float32)
    w2 = jnp.concatenate(
        [_silu(a0 * f2_ref[...] + b2_ref[...]),
         _silu(a1 * f2_ref[...] + b2_ref[...])], axis=1)
    out_ref[...] = mm * w2


def _edge(attr2, gsd, f1t, b1t, f2t, b2t, w192):
    return pl.pallas_call(
        _edge_body,
        grid=(_EP // 2 // _BE2,),
        in_specs=[
            pl.BlockSpec((_BE2, 2), lambda i: (i, 0)),
            pl.BlockSpec((_BE2, 192), lambda i: (i, 0)),
            pl.BlockSpec((1, 96), lambda i: (0, 0)),
            pl.BlockSpec((1, 96), lambda i: (0, 0)),
            pl.BlockSpec((1, 48), lambda i: (0, 0)),
            pl.BlockSpec((1, 48), lambda i: (0, 0)),
            pl.BlockSpec((192, 96), lambda i: (0, 0)),
        ],
        out_specs=pl.BlockSpec((_BE2, 96), lambda i: (i, 0)),
        out_shape=jax.ShapeDtypeStruct((_EP // 2, 96), jnp.float32),
    )(attr2, gsd, f1t, b1t, f2t, b2t, w192)


def _final_body(a1_ref, a2_ref, cd_ref, out_ref):
    a1 = a1_ref[...]
    a2 = a2_ref[...]
    bn = a1.shape[0]
    t = (jnp.dot(a1.reshape(bn * D, V), cd_ref[0:V, :],
                 preferred_element_type=jnp.float32)
         + jnp.dot(a2.reshape(bn * D, V), cd_ref[V:2 * V, :],
                   preferred_element_type=jnp.float32)).reshape(bn, D, V)
    nrm = jnp.sqrt(jnp.sum(t * t, axis=1))
    out_ref[...] = t * jnp.tanh(nrm)[:, None, :]


_BNF = 2000


def _final(acc1, acc2, cd):
    return pl.pallas_call(
        _final_body,
        grid=(N // _BNF,),
        in_specs=[
            pl.BlockSpec((_BNF, D, V), lambda i: (i, 0, 0)),
            pl.BlockSpec((_BNF, D, V), lambda i: (i, 0, 0)),
            pl.BlockSpec((2 * V, V), lambda i: (0, 0)),
        ],
        out_specs=pl.BlockSpec((_BNF, D, V), lambda i: (i, 0, 0)),
        out_shape=jax.ShapeDtypeStruct((N, D, V), jnp.float32),
    )(acc1, acc2, cd)


def kernel(xn, xn_attr, xe_attr, xe_src, xe_dst, Ms_M, Ms_w, fc1_w, fc1_b,
           fc2_w, fc2_b, n2e_M1, n2e_M2, mvv_M1, mvv_M2, e2n_M1, e2n_M2):
    ang = 0.1 * Ms_w[0]
    cw = jnp.stack([jnp.cos(ang), jnp.sin(ang)])
    Rm = (mvv_M1 + mvv_M2) * 0.5
    A = (n2e_M1 * 0.5 + n2e_M2 * 0.25) @ Rm
    B = (-n2e_M1 * 0.5 + n2e_M2 * 0.25) @ Rm
    eye3 = jnp.eye(3, dtype=jnp.float32)
    bda = jnp.kron(eye3, A)
    bdb = jnp.kron(eye3, B)
    w96 = jnp.concatenate([bda, bdb], axis=0)
    w192 = jnp.kron(jnp.eye(2, dtype=jnp.float32), w96)
    cd = jnp.concatenate([(e2n_M1 + e2n_M2) * 0.5,
                          (e2n_M2 - e2n_M1) * 0.5], axis=0)
    f1t = jnp.tile(fc1_w[:, 0], 6)[None, :]
    b1t = jnp.tile(fc1_b, 6)[None, :]
    f2t = jnp.tile(fc2_w[:, 0], 3)[None, :]
    b2t = jnp.tile(fc2_b, 3)[None, :]

    xn_p = jnp.concatenate(
        [xn, jnp.zeros((_NP - N, D, V), jnp.float32)], axis=0)
    attr_np = jnp.concatenate(
        [xn_attr, jnp.zeros((_NP - N, V), jnp.float32)], axis=0)
    xnp = _premix(xn_p, attr_np, Ms_M, cw).reshape(_NP, D * V)
    xnp_sc = _relayout_sc(xnp)

    ii = jnp.stack([xe_src, xe_dst], axis=1).reshape(2 * E)
    pad = jnp.arange(2 * _EP - 2 * E, dtype=jnp.int32) % N
    ii3 = jnp.concatenate([ii, pad]).reshape(_IW, 128)
    gsd = _gather_sc(xnp_sc, ii3).reshape(_EP // 2, 4 * D * V)

    attr_p = jnp.concatenate(
        [xe_attr, jnp.zeros((_EP - E,), jnp.float32)]).reshape(_EP // 2, 2)
    m2 = _edge(attr_p, gsd, f1t, b1t, f2t, b2t, w192)

    ipad = jnp.full((_EP - E,), 1000000, jnp.int32)
    dst3 = jnp.concatenate([xe_dst, ipad]).reshape(_SW, 128)
    src3 = jnp.concatenate([xe_src, ipad]).reshape(_SW, 128)
    m3 = m2.reshape(_SW, 128, 48)
    zeros = jnp.zeros(((_NR + _TRASH) // 16, 48), jnp.float32)
    acc1p, acc2p = _scatter_sc(m3, dst3, src3, zeros)

    acc1 = acc1p[:N].reshape(N, D, V)
    acc2 = acc2p[:N].reshape(N, D, V)
    return _final(acc1, acc2, cd)
